# baseline (device time: 396012 ns/iter reference)
import jax
import jax.numpy as jnp
from jax import lax
from jax.experimental import pallas as pl
from jax.experimental.pallas import tpu as pltpu

N_DEV = 16
NHOP = N_DEV - 1
NPIECE = 4


def kernel(x):
    m_per, n = x.shape
    half = m_per // 2
    piece = half // NPIECE

    def body(x_ref, out_ref, fs_sems, fr_sems, rs_sems, rr_sems, copy_sem):
        my = lax.axis_index("i")
        left = lax.rem(my - 1 + N_DEV, N_DEV)
        right = lax.rem(my + 1, N_DEV)

        barrier_sem = pltpu.get_barrier_semaphore()
        for nbr in (left, right):
            pl.semaphore_signal(
                barrier_sem, inc=1,
                device_id=(nbr,), device_id_type=pl.DeviceIdType.MESH,
            )
        pl.semaphore_wait(barrier_sem, 2)

        cp = pltpu.make_async_copy(
            x_ref, out_ref.at[pl.ds(my * m_per, m_per), :], copy_sem
        )
        cp.start()
        cp.wait()

        def fwd_rdma(h, p):
            c = lax.rem(my - h + N_DEV, N_DEV)
            off = c * m_per + p * piece
            return pltpu.make_async_remote_copy(
                src_ref=out_ref.at[pl.ds(off, piece), :],
                dst_ref=out_ref.at[pl.ds(off, piece), :],
                send_sem=fs_sems.at[h, p],
                recv_sem=fr_sems.at[h, p],
                device_id=(right,),
                device_id_type=pl.DeviceIdType.MESH,
            )

        def rev_rdma(h, p):
            c = lax.rem(my + h, N_DEV)
            off = c * m_per + half + p * piece
            return pltpu.make_async_remote_copy(
                src_ref=out_ref.at[pl.ds(off, piece), :],
                dst_ref=out_ref.at[pl.ds(off, piece), :],
                send_sem=rs_sems.at[h, p],
                recv_sem=rr_sems.at[h, p],
                device_id=(left,),
                device_id_type=pl.DeviceIdType.MESH,
            )

        for p in range(NPIECE):
            fwd_rdma(0, p).start()
            rev_rdma(0, p).start()

        for h in range(1, NHOP):
            for p in range(NPIECE):
                fwd_rdma(h - 1, p).wait_recv()
                fwd_rdma(h, p).start()
                rev_rdma(h - 1, p).wait_recv()
                rev_rdma(h, p).start()

        for p in range(NPIECE):
            fwd_rdma(NHOP - 1, p).wait_recv()
            rev_rdma(NHOP - 1, p).wait_recv()
        for h in range(NHOP):
            for p in range(NPIECE):
                fwd_rdma(h, p).wait_send()
                rev_rdma(h, p).wait_send()

    return pl.pallas_call(
        body,
        out_shape=jax.ShapeDtypeStruct((N_DEV * m_per, n), x.dtype),
        in_specs=[pl.BlockSpec(memory_space=pltpu.VMEM)],
        out_specs=pl.BlockSpec(memory_space=pl.ANY),
        scratch_shapes=[
            pltpu.SemaphoreType.DMA((NHOP, NPIECE)),
            pltpu.SemaphoreType.DMA((NHOP, NPIECE)),
            pltpu.SemaphoreType.DMA((NHOP, NPIECE)),
            pltpu.SemaphoreType.DMA((NHOP, NPIECE)),
            pltpu.SemaphoreType.DMA,
        ],
        compiler_params=pltpu.CompilerParams(
            collective_id=0,
            vmem_limit_bytes=100 * 1024 * 1024,
        ),
    )(x)


# device time: 392643 ns/iter; 1.0086x vs baseline; 1.0086x over previous
import jax
import jax.numpy as jnp
from jax import lax
from jax.experimental import pallas as pl
from jax.experimental.pallas import tpu as pltpu

N_DEV = 16
NHOP = N_DEV - 1
NPIECE = 2


def kernel(x):
    m_per, n = x.shape
    half = m_per // 2
    piece = half // NPIECE

    def body(x_ref, out_ref, fs_sems, fr_sems, rs_sems, rr_sems, copy_sem):
        my = lax.axis_index("i")
        left = lax.rem(my - 1 + N_DEV, N_DEV)
        right = lax.rem(my + 1, N_DEV)

        barrier_sem = pltpu.get_barrier_semaphore()
        for nbr in (left, right):
            pl.semaphore_signal(
                barrier_sem, inc=1,
                device_id=(nbr,), device_id_type=pl.DeviceIdType.MESH,
            )
        pl.semaphore_wait(barrier_sem, 2)

        cp = pltpu.make_async_copy(
            x_ref, out_ref.at[pl.ds(my * m_per, m_per), :], copy_sem
        )
        cp.start()

        def fwd_rdma(h, p):
            c = lax.rem(my - h + N_DEV, N_DEV)
            off = c * m_per + p * piece
            src = (
                x_ref.at[pl.ds(p * piece, piece), :]
                if h == 0
                else out_ref.at[pl.ds(off, piece), :]
            )
            return pltpu.make_async_remote_copy(
                src_ref=src,
                dst_ref=out_ref.at[pl.ds(off, piece), :],
                send_sem=fs_sems.at[h, p],
                recv_sem=fr_sems.at[h, p],
                device_id=(right,),
                device_id_type=pl.DeviceIdType.MESH,
            )

        def rev_rdma(h, p):
            c = lax.rem(my + h, N_DEV)
            off = c * m_per + half + p * piece
            src = (
                x_ref.at[pl.ds(half + p * piece, piece), :]
                if h == 0
                else out_ref.at[pl.ds(off, piece), :]
            )
            return pltpu.make_async_remote_copy(
                src_ref=src,
                dst_ref=out_ref.at[pl.ds(off, piece), :],
                send_sem=rs_sems.at[h, p],
                recv_sem=rr_sems.at[h, p],
                device_id=(left,),
                device_id_type=pl.DeviceIdType.MESH,
            )

        for p in range(NPIECE):
            fwd_rdma(0, p).start()
            rev_rdma(0, p).start()

        for h in range(1, NHOP):
            for p in range(NPIECE):
                fwd_rdma(h - 1, p).wait_recv()
                fwd_rdma(h, p).start()
                rev_rdma(h - 1, p).wait_recv()
                rev_rdma(h, p).start()

        for p in range(NPIECE):
            fwd_rdma(NHOP - 1, p).wait_recv()
            rev_rdma(NHOP - 1, p).wait_recv()
        for h in range(NHOP):
            for p in range(NPIECE):
                fwd_rdma(h, p).wait_send()
                rev_rdma(h, p).wait_send()
        cp.wait()

    return pl.pallas_call(
        body,
        out_shape=jax.ShapeDtypeStruct((N_DEV * m_per, n), x.dtype),
        in_specs=[pl.BlockSpec(memory_space=pltpu.VMEM)],
        out_specs=pl.BlockSpec(memory_space=pl.ANY),
        scratch_shapes=[
            pltpu.SemaphoreType.DMA((NHOP, NPIECE)),
            pltpu.SemaphoreType.DMA((NHOP, NPIECE)),
            pltpu.SemaphoreType.DMA((NHOP, NPIECE)),
            pltpu.SemaphoreType.DMA((NHOP, NPIECE)),
            pltpu.SemaphoreType.DMA,
        ],
        compiler_params=pltpu.CompilerParams(
            collective_id=0,
            vmem_limit_bytes=100 * 1024 * 1024,
        ),
    )(x)


# device time: 327119 ns/iter; 1.2106x vs baseline; 1.2003x over previous
import jax
import jax.numpy as jnp
from jax import lax
from jax.experimental import pallas as pl
from jax.experimental.pallas import tpu as pltpu

N_DEV = 16
NZ = 4
NP = 4
DELTAS = (0, 1, -1, 2, -2, 3, -3)


def kernel(x):
    m_per, n = x.shape
    half = m_per // 2

    def body(x_ref, out_ref, cwS, cwR, ccS, ccR, upS, upR, dnS, dnR, copy_sem):
        my = lax.axis_index("i")
        mz = my // NP
        mp = lax.rem(my, NP)
        pr = mz * NP + lax.rem(mp + 1, NP)
        pll = mz * NP + lax.rem(mp + 3, NP)
        up = my + NP
        dn = my - NP

        def zok(j):
            d = DELTAS[j]
            return (mz + d >= 0) & (mz + d <= NZ - 1)

        barrier_sem = pltpu.get_barrier_semaphore()
        for nbr in (pr, pll):
            pl.semaphore_signal(
                barrier_sem, inc=1,
                device_id=(nbr,), device_id_type=pl.DeviceIdType.MESH,
            )

        @pl.when(mz < NZ - 1)
        def _():
            pl.semaphore_signal(
                barrier_sem, inc=1,
                device_id=(up,), device_id_type=pl.DeviceIdType.MESH,
            )

        @pl.when(mz > 0)
        def _():
            pl.semaphore_signal(
                barrier_sem, inc=1,
                device_id=(dn,), device_id_type=pl.DeviceIdType.MESH,
            )

        pl.semaphore_wait(barrier_sem, 3)

        @pl.when((mz > 0) & (mz < NZ - 1))
        def _():
            pl.semaphore_wait(barrier_sem, 1)

        cp = pltpu.make_async_copy(
            x_ref, out_ref.at[pl.ds(my * m_per, m_per), :], copy_sem
        )
        cp.start()


        def cw_send(h, j):
            c = lax.rem(mp - h + NP, NP)
            off = ((mz + DELTAS[j]) * NP + c) * m_per
            src = (
                x_ref.at[pl.ds(0, half), :]
                if (h == 0 and j == 0)
                else out_ref.at[pl.ds(off, half), :]
            )
            return pltpu.make_async_remote_copy(
                src_ref=src,
                dst_ref=out_ref.at[pl.ds(off, half), :],
                send_sem=cwS.at[h, j],
                recv_sem=cwR.at[h, j],
                device_id=(pr,),
                device_id_type=pl.DeviceIdType.MESH,
            )

        def cw_recv(h, j):
            c = lax.rem(mp - 1 - h + NP, NP)
            off = ((mz + DELTAS[j]) * NP + c) * m_per
            ref = out_ref.at[pl.ds(off, half), :]
            return pltpu.make_async_remote_copy(
                src_ref=ref, dst_ref=ref,
                send_sem=cwS.at[h, j], recv_sem=cwR.at[h, j],
                device_id=(pr,), device_id_type=pl.DeviceIdType.MESH,
            )

        def ccw_send(h, j):
            c = lax.rem(mp + h, NP)
            off = ((mz + DELTAS[j]) * NP + c) * m_per + half
            src = (
                x_ref.at[pl.ds(half, half), :]
                if (h == 0 and j == 0)
                else out_ref.at[pl.ds(off, half), :]
            )
            return pltpu.make_async_remote_copy(
                src_ref=src,
                dst_ref=out_ref.at[pl.ds(off, half), :],
                send_sem=ccS.at[h, j],
                recv_sem=ccR.at[h, j],
                device_id=(pll,),
                device_id_type=pl.DeviceIdType.MESH,
            )

        def ccw_recv(h, j):
            c = lax.rem(mp + 1 + h, NP)
            off = ((mz + DELTAS[j]) * NP + c) * m_per + half
            ref = out_ref.at[pl.ds(off, half), :]
            return pltpu.make_async_remote_copy(
                src_ref=ref, dst_ref=ref,
                send_sem=ccS.at[h, j], recv_sem=ccR.at[h, j],
                device_id=(pll,), device_id_type=pl.DeviceIdType.MESH,
            )

        def up_send(s):
            off = ((mz - s) * NP + mp) * m_per
            src = x_ref if s == 0 else out_ref.at[pl.ds(off, m_per), :]
            return pltpu.make_async_remote_copy(
                src_ref=src,
                dst_ref=out_ref.at[pl.ds(off, m_per), :],
                send_sem=upS.at[s], recv_sem=upR.at[s],
                device_id=(up,), device_id_type=pl.DeviceIdType.MESH,
            )

        def up_recv(s):
            off = ((mz - 1 - s) * NP + mp) * m_per
            ref = out_ref.at[pl.ds(off, m_per), :]
            return pltpu.make_async_remote_copy(
                src_ref=ref, dst_ref=ref,
                send_sem=upS.at[s], recv_sem=upR.at[s],
                device_id=(up,), device_id_type=pl.DeviceIdType.MESH,
            )

        def dn_send(s):
            off = ((mz + s) * NP + mp) * m_per
            src = x_ref if s == 0 else out_ref.at[pl.ds(off, m_per), :]
            return pltpu.make_async_remote_copy(
                src_ref=src,
                dst_ref=out_ref.at[pl.ds(off, m_per), :],
                send_sem=dnS.at[s], recv_sem=dnR.at[s],
                device_id=(dn,), device_id_type=pl.DeviceIdType.MESH,
            )

        def dn_recv(s):
            off = ((mz + 1 + s) * NP + mp) * m_per
            ref = out_ref.at[pl.ds(off, m_per), :]
            return pltpu.make_async_remote_copy(
                src_ref=ref, dst_ref=ref,
                send_sem=dnS.at[s], recv_sem=dnR.at[s],
                device_id=(dn,), device_id_type=pl.DeviceIdType.MESH,
            )


        def e_up(s):
            @pl.when(mz >= s + 1)
            def _():
                up_recv(s).wait_recv()
                if s + 1 <= NZ - 2:
                    @pl.when(mz < NZ - 1)
                    def _():
                        up_send(s + 1).start()
                cw_send(0, 2 * (s + 1)).start()
                ccw_send(0, 2 * (s + 1)).start()

        def e_dn(s):
            @pl.when(mz <= NZ - 2 - s)
            def _():
                dn_recv(s).wait_recv()
                if s + 1 <= NZ - 2:
                    @pl.when(mz > 0)
                    def _():
                        dn_send(s + 1).start()
                cw_send(0, 2 * s + 1).start()
                ccw_send(0, 2 * s + 1).start()

        def e_cw(h, j):
            @pl.when(zok(j))
            def _():
                cw_recv(h, j).wait_recv()
                if h + 1 <= 2:
                    cw_send(h + 1, j).start()

        def e_ccw(h, j):
            @pl.when(zok(j))
            def _():
                ccw_recv(h, j).wait_recv()
                if h + 1 <= 2:
                    ccw_send(h + 1, j).start()


        cw_send(0, 0).start()
        ccw_send(0, 0).start()

        @pl.when(mz < NZ - 1)
        def _():
            up_send(0).start()

        @pl.when(mz > 0)
        def _():
            dn_send(0).start()

        e_cw(0, 0); e_ccw(0, 0)
        e_up(0); e_dn(0)
        e_cw(1, 0); e_ccw(1, 0)
        e_cw(0, 1); e_cw(0, 2); e_ccw(0, 1); e_ccw(0, 2)
        e_up(1); e_dn(1)
        e_cw(1, 1); e_cw(1, 2); e_ccw(1, 1); e_ccw(1, 2)
        e_cw(2, 0); e_ccw(2, 0)
        e_cw(0, 3); e_cw(0, 4); e_ccw(0, 3); e_ccw(0, 4)
        e_up(2); e_dn(2)
        e_cw(1, 3); e_cw(1, 4); e_ccw(1, 3); e_ccw(1, 4)
        e_cw(2, 1); e_cw(2, 2); e_ccw(2, 1); e_ccw(2, 2)
        e_cw(0, 5); e_cw(0, 6); e_ccw(0, 5); e_ccw(0, 6)
        e_cw(1, 5); e_cw(1, 6); e_ccw(1, 5); e_ccw(1, 6)
        for j in (3, 4, 5, 6):
            e_cw(2, j)
            e_ccw(2, j)

        for h in range(3):
            for j in range(7):
                @pl.when(zok(j))
                def _():
                    cw_send(h, j).wait_send()
                    ccw_send(h, j).wait_send()
        for s in range(NZ - 1):
            @pl.when((mz < NZ - 1) & (mz >= s))
            def _():
                up_send(s).wait_send()

            @pl.when((mz > 0) & (mz <= NZ - 1 - s))
            def _():
                dn_send(s).wait_send()
        cp.wait()

    return pl.pallas_call(
        body,
        out_shape=jax.ShapeDtypeStruct((N_DEV * m_per, n), x.dtype),
        in_specs=[pl.BlockSpec(memory_space=pltpu.VMEM)],
        out_specs=pl.BlockSpec(memory_space=pl.ANY),
        scratch_shapes=[
            pltpu.SemaphoreType.DMA((3, 7)),
            pltpu.SemaphoreType.DMA((3, 7)),
            pltpu.SemaphoreType.DMA((3, 7)),
            pltpu.SemaphoreType.DMA((3, 7)),
            pltpu.SemaphoreType.DMA((NZ - 1,)),
            pltpu.SemaphoreType.DMA((NZ - 1,)),
            pltpu.SemaphoreType.DMA((NZ - 1,)),
            pltpu.SemaphoreType.DMA((NZ - 1,)),
            pltpu.SemaphoreType.DMA,
        ],
        compiler_params=pltpu.CompilerParams(
            collective_id=0,
            vmem_limit_bytes=100 * 1024 * 1024,
        ),
    )(x)
